# baseline (speedup 1.0000x reference)
import jax
import jax.numpy as jnp
from jax import lax
from jax.experimental import pallas as pl
from jax.experimental.pallas import tpu as pltpu

_CAPACITY_FACTOR = 1.25
_MIN_CAPACITY = 4
_EPS = 1e-9
_THRESH1 = 0.2

_INTERPRET = False


def _routing_body(cap, nb, x_ref, w_ref, p1_ref,
                  flat0_ref, g1_ref, i2_ref, r1m_ref, g2_ref, stats_ref):
    t = x_ref.shape[1]
    e = w_ref.shape[1]
    j = pl.program_id(1)
    cap_f = float(cap)

    @pl.when(j == 0)
    def _init():
        stats_ref[...] = jnp.zeros_like(stats_ref)

    xb = x_ref[0]
    logits = jnp.dot(xb, w_ref[...], preferred_element_type=jnp.float32)
    m = jnp.max(logits, axis=-1, keepdims=True)
    ex = jnp.exp(logits - m)
    s = jnp.sum(ex, axis=-1, keepdims=True)
    raw = ex / s
    lse = jnp.log(s) + m
    zblk = jnp.sum(lse * lse)

    eidx = lax.broadcasted_iota(jnp.int32, (t, e), 1)
    m1 = jnp.max(raw, axis=-1, keepdims=True)
    i1 = jnp.min(jnp.where(raw == m1, eidx, e), axis=-1, keepdims=True)
    raw2 = jnp.where(eidx == i1, -1.0, raw)
    m2 = jnp.max(raw2, axis=-1, keepdims=True)
    i2 = jnp.min(jnp.where(raw2 == m2, eidx, e), axis=-1, keepdims=True)

    denom = jnp.maximum(m1 + m2, _EPS)
    g1n = m1 / denom
    g2n = m2 / denom
    p1 = p1_ref[0]
    route1 = p1 < (g2n / _THRESH1)

    mask0 = (eidx == i1).astype(jnp.float32)
    mask1 = (eidx == i2).astype(jnp.float32) * route1.astype(jnp.float32)

    ti = lax.broadcasted_iota(jnp.int32, (t, t), 0)
    tj = lax.broadcasted_iota(jnp.int32, (t, t), 1)
    tri = (tj < ti).astype(jnp.float32)
    excl0 = jnp.dot(tri, mask0, preferred_element_type=jnp.float32)
    excl1 = jnp.dot(tri, mask1, preferred_element_type=jnp.float32)

    sts = stats_ref[...]
    prev0 = sts[0, 0:1, :]
    prev1 = sts[0, 3:4, :]

    rank0 = jnp.sum((excl0 + prev0) * mask0, axis=-1, keepdims=True)
    rank1 = jnp.sum((excl1 + prev1) * mask1, axis=-1, keepdims=True)
    acc0 = rank0 < cap_f
    flat0 = jnp.where(acc0, i1 * cap + rank0.astype(jnp.int32), -1)
    r1m = jnp.where(route1, rank1, 1e9)

    bsum0 = jnp.sum(mask0, axis=0, keepdims=True)
    bsum1 = jnp.sum(mask1, axis=0, keepdims=True)
    braw = jnp.sum(raw, axis=0, keepdims=True)
    riota = lax.broadcasted_iota(jnp.int32, (1, 8, e), 1)
    liota = lax.broadcasted_iota(jnp.int32, (1, 8, e), 2)
    delta = (jnp.where(riota == 0, bsum0[None], 0.0)
             + jnp.where(riota == 1, braw[None], 0.0)
             + jnp.where(riota == 3, bsum1[None], 0.0)
             + jnp.where((riota == 2) & (liota == 0), zblk, 0.0))
    stats_ref[...] = sts + delta

    flat0_ref[...] = flat0[None]
    g1_ref[...] = g1n[None]
    i2_ref[...] = i2[None]
    r1m_ref[...] = r1m[None]
    g2_ref[...] = g2n[None]


def _materialize_body(cap, flat0_ref, g1_ref, i2_ref, r1m_ref, g2_ref,
                      stats_ref, comb_ref, disp_ref):
    t = comb_ref.shape[1]
    ec = comb_ref.shape[2]
    e = stats_ref.shape[2]
    cap_f = float(cap)

    total0 = stats_ref[0, 0:1, :]
    count0 = jnp.minimum(total0, cap_f)
    i2t = i2_ref[0]
    eidx = lax.broadcasted_iota(jnp.int32, (t, e), 1)
    cnt = jnp.sum((eidx == i2t).astype(jnp.float32) * count0,
                  axis=-1, keepdims=True)
    pos1 = r1m_ref[0] + cnt
    acc1 = pos1 < cap_f
    flat1 = jnp.where(acc1, i2t * cap + pos1.astype(jnp.int32), -1)

    lane = lax.broadcasted_iota(jnp.int32, (t, ec), 1)
    comb = (jnp.where(lane == flat0_ref[0], g1_ref[0], 0.0)
            + jnp.where(lane == flat1, g2_ref[0], 0.0))
    comb_ref[...] = comb[None]
    disp_ref[...] = (comb != 0.0).astype(jnp.float32)[None]


def kernel(x, W):
    b, n, d = x.shape
    e = W.shape[1]
    cap = min(n, int(n * _CAPACITY_FACTOR / e))
    cap = max(cap, _MIN_CAPACITY)
    ec = e * cap

    t1 = min(512, n)
    nb1 = n // t1
    t2 = min(256, n)
    nb2 = n // t2

    probs = jax.random.uniform(jax.random.key(42), (2, b, n),
                               dtype=jnp.float32)
    p1 = probs[1].reshape(b, n, 1)

    tok = lambda dt: jax.ShapeDtypeStruct((b, n, 1), dt)
    tok_spec1 = pl.BlockSpec((1, t1, 1), lambda i, j: (i, j, 0))
    stats_spec = pl.BlockSpec((1, 8, e), lambda i, j: (i, 0, 0))

    flat0, g1, i2, r1m, g2, stats = pl.pallas_call(
        lambda *refs: _routing_body(cap, nb1, *refs),
        grid=(b, nb1),
        in_specs=[
            pl.BlockSpec((1, t1, d), lambda i, j: (i, j, 0)),
            pl.BlockSpec((d, e), lambda i, j: (0, 0)),
            tok_spec1,
        ],
        out_specs=[tok_spec1, tok_spec1, tok_spec1, tok_spec1, tok_spec1,
                   stats_spec],
        out_shape=[tok(jnp.int32), tok(jnp.float32), tok(jnp.int32),
                   tok(jnp.float32), tok(jnp.float32),
                   jax.ShapeDtypeStruct((b, 8, e), jnp.float32)],
        interpret=_INTERPRET,
    )(x, W, p1)

    tok_spec2 = pl.BlockSpec((1, t2, 1), lambda i, j: (i, j, 0))
    big_spec = pl.BlockSpec((1, t2, ec), lambda i, j: (i, j, 0))
    comb, disp = pl.pallas_call(
        lambda *refs: _materialize_body(cap, *refs),
        grid=(b, nb2),
        in_specs=[tok_spec2, tok_spec2, tok_spec2, tok_spec2, tok_spec2,
                  stats_spec],
        out_specs=[big_spec, big_spec],
        out_shape=[jax.ShapeDtypeStruct((b, n, ec), jnp.float32),
                   jax.ShapeDtypeStruct((b, n, ec), jnp.float32)],
        interpret=_INTERPRET,
    )(flat0, g1, i2, r1m, g2, stats)

    combine_tensor = comb.reshape(b, n, e, cap)
    dispatch_tensor = disp.reshape(b, n, e, cap).astype(x.dtype)

    density_1 = stats[:, 0, :] / n
    density_proxy = stats[:, 1, :] / n
    balance_loss = jnp.mean(density_proxy * density_1) * float(e * e)
    router_z_loss = jnp.sum(stats[:, 2, 0]) / (b * n)

    return (dispatch_tensor, combine_tensor, balance_loss, router_z_loss)
